# Initial kernel scaffold; baseline (speedup 1.0000x reference)
import jax
import jax.numpy as jnp
from jax.experimental import pallas as pl


def kernel(x_paper, x_author, c1, c2, ei_pp, ei_aa, ei_ap, ei_pa):
    raise NotImplementedError("write your pallas kernel here")


# speedup vs baseline: 7.2974x; 7.2974x over previous
import functools

import jax
import jax.numpy as jnp
from jax import lax
from jax.experimental import pallas as pl
from jax.experimental.pallas import tpu as pltpu
from jax.experimental.pallas import tpu_sc as plsc

N = 50000
E = 150000
D = 128
EB = 128
EP = ((E + EB - 1) // EB) * EB
NBATCH = EP // EB
BLK = 2000
HALF = N // 2
CH = 4
CSZ = 6256
ACC_ROWS = 6400
DUMMY = CSZ + 8


def _proj_body(heads, x_ref, w_ref, b_ref, av_ref, h_ref, att_ref):
    dh = D // heads
    h = jnp.dot(x_ref[...], w_ref[...], preferred_element_type=jnp.float32)
    h = h + b_ref[...]
    h_ref[...] = h
    cols = []
    for t in range(4):
        prod = h * av_ref[t:t + 1, :]
        per_head = [prod[:, k * dh:(k + 1) * dh].sum(-1, keepdims=True)
                    for k in range(heads)]
        per_head.append(jnp.zeros((h.shape[0], 16 - heads), jnp.float32))
        cols.append(jnp.concatenate(per_head, axis=1))
    att_ref[...] = jnp.concatenate(cols, axis=1)


def _proj(x, w, b, av, heads):
    grid = (N // BLK,)
    return pl.pallas_call(
        functools.partial(_proj_body, heads),
        grid=grid,
        in_specs=[
            pl.BlockSpec((BLK, D), lambda i: (i, 0)),
            pl.BlockSpec((D, D), lambda i: (0, 0)),
            pl.BlockSpec((1, D), lambda i: (0, 0)),
            pl.BlockSpec((4, D), lambda i: (0, 0)),
        ],
        out_specs=[
            pl.BlockSpec((BLK, D), lambda i: (i, 0)),
            pl.BlockSpec((BLK, 64), lambda i: (i, 0)),
        ],
        out_shape=[
            jax.ShapeDtypeStruct((N, D), jnp.float32),
            jax.ShapeDtypeStruct((N, 64), jnp.float32),
        ],
    )(x, w, b.reshape(1, D), av)


def _group_stats_body(o1_ref, o2_ref, kw_ref, kb_ref, out_ref):
    i = pl.program_id(0)
    kw = kw_ref[...]
    kb = kb_ref[...]
    t1 = jnp.tanh(jnp.dot(o1_ref[...], kw, preferred_element_type=jnp.float32)
                  + kb).sum(0, keepdims=True)
    t2 = jnp.tanh(jnp.dot(o2_ref[...], kw, preferred_element_type=jnp.float32)
                  + kb).sum(0, keepdims=True)
    part = jnp.concatenate([t1, t2], axis=0)

    @pl.when(i == 0)
    def _():
        out_ref[...] = part

    @pl.when(i > 0)
    def _():
        out_ref[...] = out_ref[...] + part


def _group_stats(o1, o2, kw, kb):
    return pl.pallas_call(
        _group_stats_body,
        grid=(N // BLK,),
        in_specs=[
            pl.BlockSpec((BLK, D), lambda i: (i, 0)),
            pl.BlockSpec((BLK, D), lambda i: (i, 0)),
            pl.BlockSpec((D, D), lambda i: (0, 0)),
            pl.BlockSpec((1, D), lambda i: (0, 0)),
        ],
        out_specs=pl.BlockSpec((2, D), lambda i: (0, 0)),
        out_shape=jax.ShapeDtypeStruct((2, D), jnp.float32),
    )(o1, o2, kw, kb.reshape(1, D))


def _combine_body(elu, o1_ref, o2_ref, w_ref, out_ref):
    w = w_ref[...]
    y = o1_ref[...] * w[0:1, :] + o2_ref[...] * w[1:2, :]
    if elu:
        y = jnp.where(y > 0, y, jnp.exp(y) - 1.0)
    out_ref[...] = y


def _combine(o1, o2, wvec, elu):
    w = jnp.broadcast_to(wvec.reshape(2, 1), (2, D))
    return pl.pallas_call(
        functools.partial(_combine_body, elu),
        grid=(N // BLK,),
        in_specs=[
            pl.BlockSpec((BLK, D), lambda i: (i, 0)),
            pl.BlockSpec((BLK, D), lambda i: (i, 0)),
            pl.BlockSpec((2, D), lambda i: (0, 0)),
        ],
        out_specs=pl.BlockSpec((BLK, D), lambda i: (i, 0)),
        out_shape=jax.ShapeDtypeStruct((N, D), jnp.float32),
    )(o1, o2, w)


def _splat(vec, lane):
    idx = jnp.full((16,), lane, jnp.int32)
    return vec.at[idx].get(mode="promise_in_bounds")


def _sc_edge_body(heads, hs_hbm, ats_hbm, atd_hbm, ei_hbm, out_hbm,
                  s_idx, d_idx, dl, asrc_r, adst_r, ex16, rows,
                  zbuf, zd, fin, fden, acc, dacc, sem, sem2):
    dh = D // heads
    hv = [(16 * v) // dh for v in range(8)]
    c = lax.axis_index("c")
    s = lax.axis_index("s")
    zero16 = jnp.zeros((16,), jnp.float32)

    def _z(i, _):
        r = i // 8
        j = i % 8
        zbuf[r, pl.ds(j * 16, 16)] = zero16
        return 0
    lax.fori_loop(0, 1024, _z, 0)

    def _zd(i, _):
        zd[i, :] = zero16
        return 0
    lax.fori_loop(0, 128, _zd, 0)

    for k in range(CH):
        lo = c * HALF + k * CSZ
        sz = min(CSZ, HALF - k * CSZ)

        per_tile = ACC_ROWS // 16
        r0 = s * per_tile
        off = 0
        while off < per_tile:
            n = min(128, per_tile - off)
            pltpu.sync_copy(zbuf.at[pl.ds(0, n), :],
                            acc.at[pl.ds(r0 + off, n), :])
            pltpu.sync_copy(zd.at[pl.ds(0, n), :],
                            dacc.at[pl.ds(r0 + off, n), :])
            off += n
        plsc.subcore_barrier()

        def _batch(j, _):
            b = j * 16 + s

            @pl.when(b < NBATCH)
            def _():
                base = b * EB
                pltpu.sync_copy(ei_hbm.at[0, pl.ds(base, EB)], s_idx)
                pltpu.sync_copy(ei_hbm.at[1, pl.ds(base, EB)], d_idx)
                c1 = pltpu.async_copy(hs_hbm.at[s_idx], rows, sem)
                c2 = pltpu.async_copy(ats_hbm.at[s_idx], asrc_r, sem2)
                c3 = pltpu.async_copy(atd_hbm.at[d_idx], adst_r, sem2)

                def _dl(i, _):
                    dv = d_idx[pl.ds(i * 16, 16)]
                    inb = (dv >= lo) & (dv < lo + sz)
                    dl[pl.ds(i * 16, 16)] = jnp.where(inb, dv - lo, DUMMY)
                    return 0
                lax.fori_loop(0, EB // 16, _dl, 0)

                c2.wait()
                c3.wait()

                def _ex(i, _):
                    x = asrc_r[i, :] + adst_r[i, :]
                    x = jnp.maximum(x, 0.2 * x)
                    ex16[i, :] = jnp.exp(x)
                    return 0
                lax.fori_loop(0, EB, _ex, 0)

                c1.wait()

                def _scale(i, _):
                    exv = ex16[i, :]
                    for v in range(8):
                        m = _splat(exv, hv[v])
                        rows[i, pl.ds(v * 16, 16)] = (
                            rows[i, pl.ds(v * 16, 16)] * m)
                    return 0
                lax.fori_loop(0, EB, _scale, 0)

                pltpu.sync_copy(rows, acc.at[dl], add=True)
                pltpu.sync_copy(ex16, dacc.at[dl], add=True)
            return 0
        lax.fori_loop(0, (NBATCH + 15) // 16, _batch, 0)
        plsc.subcore_barrier()

        nrb = (sz + 127) // 128
        tail = sz - (nrb - 1) * 128

        def _fin_rows(nr):
            def _fr(r, _):
                dv = fden[r, :]
                for v in range(8):
                    m = _splat(dv, hv[v])
                    x = fin[r, pl.ds(v * 16, 16)]
                    fin[r, pl.ds(v * 16, 16)] = jnp.maximum(
                        x / (m + 1e-16), 0.0)
                return 0
            lax.fori_loop(0, nr, _fr, 0)

        def _fb(j, _):
            rb = j * 16 + s
            rr = rb * 128

            @pl.when(rb < nrb - 1)
            def _():
                pltpu.sync_copy(acc.at[pl.ds(rr, 128), :], fin)
                pltpu.sync_copy(dacc.at[pl.ds(rr, 128), :], fden)
                _fin_rows(128)
                pltpu.sync_copy(fin, out_hbm.at[pl.ds(lo + rr, 128)])

            @pl.when(rb == nrb - 1)
            def _():
                pltpu.sync_copy(acc.at[pl.ds(rr, tail), :],
                                fin.at[pl.ds(0, tail), :])
                pltpu.sync_copy(dacc.at[pl.ds(rr, tail), :],
                                fden.at[pl.ds(0, tail), :])
                _fin_rows(tail)
                pltpu.sync_copy(fin.at[pl.ds(0, tail), :],
                                out_hbm.at[pl.ds(lo + rr, tail)])
            return 0
        lax.fori_loop(0, (nrb + 15) // 16, _fb, 0)
        plsc.subcore_barrier()


def _sc_edge(h_src, ats, atd, ei_pad, heads):
    mesh = plsc.VectorSubcoreMesh(core_axis_name="c", subcore_axis_name="s")
    f = pl.kernel(
        functools.partial(_sc_edge_body, heads),
        out_type=jax.ShapeDtypeStruct((N, D), jnp.float32),
        mesh=mesh,
        scratch_types=[
            pltpu.VMEM((EB,), jnp.int32),
            pltpu.VMEM((EB,), jnp.int32),
            pltpu.VMEM((EB,), jnp.int32),
            pltpu.VMEM((EB, 16), jnp.float32),
            pltpu.VMEM((EB, 16), jnp.float32),
            pltpu.VMEM((EB, 16), jnp.float32),
            pltpu.VMEM((EB, D), jnp.float32),
            pltpu.VMEM((128, D), jnp.float32),
            pltpu.VMEM((128, 16), jnp.float32),
            pltpu.VMEM((128, D), jnp.float32),
            pltpu.VMEM((128, 16), jnp.float32),
            pltpu.VMEM_SHARED((ACC_ROWS, D), jnp.float32),
            pltpu.VMEM_SHARED((ACC_ROWS, 16), jnp.float32),
            pltpu.SemaphoreType.DMA,
            pltpu.SemaphoreType.DMA,
        ],
        compiler_params=pltpu.CompilerParams(use_tc_tiling_on_sc=False),
    )
    return f(h_src, ats, atd, ei_pad)


def _pad_ei(ei):
    pad = EP - E
    ps = jnp.zeros((1, pad), jnp.int32)
    pd = jnp.full((1, pad), 1 << 29, jnp.int32)
    return jnp.concatenate(
        [ei.astype(jnp.int32), jnp.concatenate([ps, pd], axis=0)], axis=1)


def _han_layer(xp, xa, p, heads, eis):
    ei_pp, ei_aa, ei_ap, ei_pa = eis
    avp = jnp.concatenate([
        p["att_src_pp"].reshape(1, D), p["att_dst_pp"].reshape(1, D),
        p["att_dst_ap"].reshape(1, D), p["att_src_pa"].reshape(1, D)], axis=0)
    ava = jnp.concatenate([
        p["att_src_aa"].reshape(1, D), p["att_dst_aa"].reshape(1, D),
        p["att_src_ap"].reshape(1, D), p["att_dst_pa"].reshape(1, D)], axis=0)
    hp, attp = _proj(xp, p["proj_paper_w"], p["proj_paper_b"], avp, heads)
    ha, atta = _proj(xa, p["proj_author_w"], p["proj_author_b"], ava, heads)
    tp = [attp[:, i * 16:(i + 1) * 16] for i in range(4)]
    ta = [atta[:, i * 16:(i + 1) * 16] for i in range(4)]
    o_pp = _sc_edge(hp, tp[0], tp[1], ei_pp, heads)
    o_aa = _sc_edge(ha, ta[0], ta[1], ei_aa, heads)
    o_ap = _sc_edge(ha, ta[2], tp[2], ei_ap, heads)
    o_pa = _sc_edge(hp, tp[3], ta[3], ei_pa, heads)
    kw, kb, q = p["k_w"], p["k_b"], p["q"]
    sp = _group_stats(o_pp, o_ap, kw, kb)
    sa = _group_stats(o_aa, o_pa, kw, kb)
    wp = jax.nn.softmax((q[0] * (sp / N)).sum(-1))
    wa = jax.nn.softmax((q[0] * (sa / N)).sum(-1))
    return (o_pp, o_ap, wp), (o_aa, o_pa, wa)


def kernel(x_paper, x_author, c1, c2, ei_pp, ei_aa, ei_ap, ei_pa):
    eis = tuple(_pad_ei(e) for e in (ei_pp, ei_aa, ei_ap, ei_pa))
    (opp, oap, wp), (oaa, opa, wa) = _han_layer(x_paper, x_author, c1, 4, eis)
    p1 = _combine(opp, oap, wp, elu=True)
    a1 = _combine(oaa, opa, wa, elu=True)
    (opp, oap, wp), (oaa, opa, wa) = _han_layer(p1, a1, c2, 1, eis)
    p2 = _combine(opp, oap, wp, elu=False)
    a2 = _combine(oaa, opa, wa, elu=False)
    return p2, a2
